# speedup vs baseline: 5.8320x; 5.8320x over previous
import functools

import numpy as np
import jax
import jax.numpy as jnp
from jax import lax
from jax.experimental import pallas as pl
from jax.experimental.pallas import tpu as pltpu
from jax.experimental.pallas import tpu_sc as plsc

NC, NS, L = 2, 16, 16
NW = NC * NS
C = 100


def _pe_table(seq, d):
    k = np.arange(seq, dtype=np.float32)[:, None]
    i = np.arange(d // 2, dtype=np.float32)[None, :]
    ang = k / np.power(10000.0, 2.0 * i / d).astype(np.float32)
    pe = np.zeros((seq, d), dtype=np.float32)
    pe[:, 0::2] = np.sin(ang)
    pe[:, 1::2] = np.cos(ang)
    return pe


@functools.partial(jax.jit, static_argnames=("seq", "d"))
def _build(tokens, table, pe, *, seq, d):
    n = tokens.size
    nch = n // (NW * C)
    idx = tokens.reshape(NW, nch, C)

    @functools.partial(
        pl.kernel,
        out_type=jax.ShapeDtypeStruct((n // C, C, d), jnp.float32),
        mesh=plsc.VectorSubcoreMesh(core_axis_name="c", subcore_axis_name="s"),
        scratch_types=[
            pltpu.VMEM((nch, C), jnp.int32),
            pltpu.VMEM((seq, d), jnp.float32),
            pltpu.VMEM((2, C, d), jnp.float32),
            pltpu.VMEM((2, C, d), jnp.float32),
            pltpu.SemaphoreType.DMA,
            pltpu.SemaphoreType.DMA,
            pltpu.SemaphoreType.DMA,
            pltpu.SemaphoreType.DMA,
        ],
    )
    def emb(table_hbm, idx_hbm, pe_hbm, out_hbm,
            idx_v, pe_v, gbuf, sbuf, gsem0, gsem1, ssem0, ssem1):
        wid = lax.axis_index("s") * NC + lax.axis_index("c")
        pltpu.sync_copy(idx_hbm.at[wid], idx_v)
        pltpu.sync_copy(pe_hbm, pe_v)
        gsems = (gsem0, gsem1)
        ssems = (ssem0, ssem1)

        for b in range(2):
            pltpu.make_async_copy(
                table_hbm.at[idx_v.at[b]], gbuf.at[b], gsems[b]).start()

        def pass_body(k, carry):
            for b in range(2):
                j = 2 * k + b
                pltpu.make_async_copy(
                    table_hbm.at[idx_v.at[j]], gbuf.at[b], gsems[b]).wait()

                @pl.when(k > 0)
                def _():
                    pltpu.make_async_copy(
                        sbuf.at[b], out_hbm.at[0], ssems[b]).wait()

                pe_base = b * C

                @plsc.parallel_loop(0, C, unroll=4)
                def _(r):
                    for cc in range(d // L):
                        sl = pl.ds(cc * L, L)
                        sbuf[b, r, sl] = gbuf[b, r, sl] + pe_v[pe_base + r, sl]

                @pl.when(j + 2 < nch)
                def _():
                    pltpu.make_async_copy(
                        table_hbm.at[idx_v.at[j + 2]], gbuf.at[b], gsems[b]
                    ).start()

                pltpu.make_async_copy(
                    sbuf.at[b], out_hbm.at[wid * nch + j], ssems[b]).start()
            return carry

        lax.fori_loop(0, nch // 2, pass_body, 0)

        for b in range(2):
            pltpu.make_async_copy(sbuf.at[b], out_hbm.at[0], ssems[b]).wait()

    return emb(table, idx, pe)


def kernel(tokens, table):
    b, s = tokens.shape
    v, d = table.shape
    assert (b * s) % (NW * C) == 0 and s % C == 0 and d % L == 0
    pe = jnp.asarray(_pe_table(s, d))
    out = _build(tokens, table, pe, seq=s, d=d)
    return out.reshape(b, s, d)


# speedup vs baseline: 10.8825x; 1.8660x over previous
import functools

import numpy as np
import jax
import jax.numpy as jnp
from jax import lax
from jax.experimental import pallas as pl
from jax.experimental.pallas import tpu as pltpu
from jax.experimental.pallas import tpu_sc as plsc

NC, NS, L = 2, 16, 16
NW = NC * NS
C = 100


def _pe_table(seq, d):
    k = np.arange(seq, dtype=np.float32)[:, None]
    i = np.arange(d // 2, dtype=np.float32)[None, :]
    ang = k / np.power(10000.0, 2.0 * i / d).astype(np.float32)
    pe = np.zeros((seq, d), dtype=np.float32)
    pe[:, 0::2] = np.sin(ang)
    pe[:, 1::2] = np.cos(ang)
    return pe


@functools.partial(jax.jit, static_argnames=("seq", "d"))
def _build(tokens, table, pe, *, seq, d):
    n = tokens.size
    nch = n // (NW * C)
    idx = tokens.reshape(NW, nch, C)

    bs = tokens.shape[0]
    rows_per_w = bs // NW

    @functools.partial(
        pl.kernel,
        out_type=jax.ShapeDtypeStruct((bs, seq, d), jnp.float32),
        mesh=plsc.VectorSubcoreMesh(core_axis_name="c", subcore_axis_name="s"),
        scratch_types=[
            pltpu.VMEM((nch, C), jnp.int32),
            pltpu.VMEM((seq, d), jnp.float32),
            pltpu.VMEM((2, C, d), jnp.float32),
            pltpu.VMEM((2, seq, d), jnp.float32),
            pltpu.SemaphoreType.DMA,
            pltpu.SemaphoreType.DMA,
            pltpu.SemaphoreType.DMA,
            pltpu.SemaphoreType.DMA,
        ],
    )
    def emb(table_hbm, idx_hbm, pe_hbm, out_hbm,
            idx_v, pe_v, gbuf, sbuf, gsem0, gsem1, ssem0, ssem1):
        wid = lax.axis_index("s") * NC + lax.axis_index("c")
        pltpu.sync_copy(idx_hbm.at[wid], idx_v)
        pltpu.sync_copy(pe_hbm, pe_v)
        gsems = (gsem0, gsem1)
        ssems = (ssem0, ssem1)

        for b in range(2):
            pltpu.make_async_copy(
                table_hbm.at[idx_v.at[b]], gbuf.at[b], gsems[b]).start()

        def pass_body(kk, carry):
            for p in range(2):
                k = 2 * kk + p
                for b in range(2):
                    j = 2 * k + b
                    pltpu.make_async_copy(
                        table_hbm.at[idx_v.at[j]], gbuf.at[b], gsems[b]).wait()

                    if b == 0:
                        @pl.when(kk > 0)
                        def _():
                            pltpu.make_async_copy(
                                sbuf.at[p], out_hbm.at[0], ssems[p]).wait()

                    pe_base = b * C

                    @plsc.parallel_loop(0, C, unroll=4)
                    def _(r):
                        for cc in range(d // L):
                            sl = pl.ds(cc * L, L)
                            sbuf[p, pe_base + r, sl] = (
                                gbuf[b, r, sl] + pe_v[pe_base + r, sl])

                    @pl.when(j + 2 < nch)
                    def _():
                        pltpu.make_async_copy(
                            table_hbm.at[idx_v.at[j + 2]], gbuf.at[b], gsems[b]
                        ).start()

                pltpu.make_async_copy(
                    sbuf.at[p], out_hbm.at[wid * rows_per_w + k], ssems[p]
                ).start()
            return carry

        lax.fori_loop(0, rows_per_w // 2, pass_body, 0)

        for p in range(2):
            pltpu.make_async_copy(sbuf.at[p], out_hbm.at[0], ssems[p]).wait()

    return emb(table, idx, pe)


def kernel(tokens, table):
    b, s = tokens.shape
    v, d = table.shape
    assert (b * s) % (NW * C) == 0 and s % C == 0 and d % L == 0
    pe = jnp.asarray(_pe_table(s, d))
    return _build(tokens, table, pe, seq=s, d=d)


# speedup vs baseline: 11.6872x; 1.0739x over previous
import functools

import numpy as np
import jax
import jax.numpy as jnp
from jax import lax
from jax.experimental import pallas as pl
from jax.experimental.pallas import tpu as pltpu
from jax.experimental.pallas import tpu_sc as plsc

NC, NS, L = 2, 16, 16
NW = NC * NS
G = 40
NSLOT = 10


def _pe_table(seq, d):
    k = np.arange(seq, dtype=np.float32)[:, None]
    i = np.arange(d // 2, dtype=np.float32)[None, :]
    ang = k / np.power(10000.0, 2.0 * i / d).astype(np.float32)
    pe = np.zeros((seq, d), dtype=np.float32)
    pe[:, 0::2] = np.sin(ang)
    pe[:, 1::2] = np.cos(ang)
    return pe


@functools.partial(jax.jit, static_argnames=("seq", "d"))
def _build(tokens, table, pe, *, seq, d):
    n = tokens.size
    ng = n // (NW * G)
    gps = seq // G
    npass = ng // NSLOT
    bs = tokens.shape[0]
    rows_per_w = bs // NW
    idx = tokens.reshape(NW, ng, G)

    @functools.partial(
        pl.kernel,
        out_type=jax.ShapeDtypeStruct((bs, seq, d), jnp.float32),
        mesh=plsc.VectorSubcoreMesh(core_axis_name="c", subcore_axis_name="s"),
        scratch_types=[
            pltpu.VMEM((ng, G), jnp.int32),
            pltpu.VMEM((seq, d), jnp.float32),
            pltpu.VMEM((NSLOT, G, d), jnp.float32),
            [pltpu.SemaphoreType.DMA] * NSLOT,
            [pltpu.SemaphoreType.DMA] * NSLOT,
        ],
    )
    def emb(table_hbm, idx_hbm, pe_hbm, out_hbm, idx_v, pe_v, buf, gsems, ssems):
        wid = lax.axis_index("s") * NC + lax.axis_index("c")
        pltpu.sync_copy(idx_hbm.at[wid], idx_v)
        pltpu.sync_copy(pe_hbm, pe_v)
        row0 = wid * rows_per_w

        def fill_and_gather(s, t):
            pe_base = (s % gps) * G

            @plsc.parallel_loop(0, G, unroll=4)
            def _(r):
                for cc in range(d // L):
                    sl = pl.ds(cc * L, L)
                    buf[s, r, sl] = pe_v[pe_base + r, sl]

            pltpu.async_copy(
                table_hbm.at[idx_v.at[t]], buf.at[s], gsems[s], add=True)

        def start_scatter(s, k, t):
            pltpu.make_async_copy(
                buf.at[s],
                out_hbm.at[row0 + 2 * k + s // gps, pl.ds((s % gps) * G, G)],
                ssems[s]).start()

        def wait_gather(s, t):
            pltpu.make_async_copy(
                table_hbm.at[idx_v.at[t]], buf.at[s], gsems[s]).wait()

        def wait_scatter(s):
            pltpu.make_async_copy(
                buf.at[s], out_hbm.at[0, pl.ds(0, G)], ssems[s]).wait()

        def pass_body(k, carry):
            @pl.when(k > 0)
            def _():
                for s in range(NSLOT):
                    t = NSLOT * (k - 1) + s
                    wait_gather(s, t)
                    start_scatter(s, k - 1, t)

            for s in range(NSLOT):
                t = NSLOT * k + s

                @pl.when(k > 0)
                def _():
                    wait_scatter(s)

                fill_and_gather(s, t)
            return carry

        lax.fori_loop(0, npass, pass_body, 0, unroll=False)

        for s in range(NSLOT):
            t = NSLOT * (npass - 1) + s
            wait_gather(s, t)
            start_scatter(s, npass - 1, t)
        for s in range(NSLOT):
            wait_scatter(s)

    return emb(table, idx, pe)


def kernel(tokens, table):
    b, s = tokens.shape
    v, d = table.shape
    assert (b * s) % (NW * G * NSLOT) == 0 and s % G == 0 and d % L == 0
    pe = jnp.asarray(_pe_table(s, d))
    return _build(tokens, table, pe, seq=s, d=d)


# speedup vs baseline: 11.8648x; 1.0152x over previous
import functools

import numpy as np
import jax
import jax.numpy as jnp
from jax import lax
from jax.experimental import pallas as pl
from jax.experimental.pallas import tpu as pltpu
from jax.experimental.pallas import tpu_sc as plsc

NC, NS, L = 2, 16, 16
NW = NC * NS
G = 40
NSLOT = 10
KLAG = 5


def _pe_table(seq, d):
    k = np.arange(seq, dtype=np.float32)[:, None]
    i = np.arange(d // 2, dtype=np.float32)[None, :]
    ang = k / np.power(10000.0, 2.0 * i / d).astype(np.float32)
    pe = np.zeros((seq, d), dtype=np.float32)
    pe[:, 0::2] = np.sin(ang)
    pe[:, 1::2] = np.cos(ang)
    return pe


@functools.partial(jax.jit, static_argnames=("seq", "d"))
def _build(tokens, table, pe, *, seq, d):
    n = tokens.size
    ng = n // (NW * G)
    gps = seq // G
    npass = ng // NSLOT
    bs = tokens.shape[0]
    rows_per_w = bs // NW
    idx = tokens.reshape(NW, ng, G)

    @functools.partial(
        pl.kernel,
        out_type=jax.ShapeDtypeStruct((bs, seq, d), jnp.float32),
        mesh=plsc.VectorSubcoreMesh(core_axis_name="c", subcore_axis_name="s"),
        scratch_types=[
            pltpu.VMEM((ng, G), jnp.int32),
            pltpu.VMEM((seq, d), jnp.float32),
            pltpu.VMEM((NSLOT, G, d), jnp.float32),
            [pltpu.SemaphoreType.DMA] * NSLOT,
            [pltpu.SemaphoreType.DMA] * NSLOT,
        ],
    )
    def emb(table_hbm, idx_hbm, pe_hbm, out_hbm, idx_v, pe_v, buf, gsems, ssems):
        wid = lax.axis_index("s") * NC + lax.axis_index("c")
        pltpu.sync_copy(idx_hbm.at[wid], idx_v)
        pltpu.sync_copy(pe_hbm, pe_v)
        row0 = wid * rows_per_w

        def fill_and_gather(s, t):
            pe_base = (s % gps) * G

            @plsc.parallel_loop(0, G, unroll=4)
            def _(r):
                for cc in range(d // L):
                    sl = pl.ds(cc * L, L)
                    buf[s, r, sl] = pe_v[pe_base + r, sl]

            pltpu.async_copy(
                table_hbm.at[idx_v.at[t]], buf.at[s], gsems[s], add=True)

        def retire(sr, tr, row, col):
            pltpu.make_async_copy(
                table_hbm.at[idx_v.at[tr]], buf.at[sr], gsems[sr]).wait()
            pltpu.make_async_copy(
                buf.at[sr], out_hbm.at[row, pl.ds(col, G)], ssems[sr]).start()

        def wait_scatter(s):
            pltpu.make_async_copy(
                buf.at[s], out_hbm.at[0, pl.ds(0, G)], ssems[s]).wait()

        def pass_body(k, carry):
            for s in range(NSLOT):
                t = NSLOT * k + s

                @pl.when(k > 0)
                def _():
                    wait_scatter(s)

                fill_and_gather(s, t)

                sr = (s + NSLOT - KLAG) % NSLOT
                tr = t - KLAG
                if s >= KLAG:
                    row = row0 + 2 * k + (s - KLAG) // gps
                    col = ((s - KLAG) % gps) * G
                    retire(sr, tr, row, col)
                else:
                    row = row0 + 2 * (k - 1) + (s + KLAG) // gps
                    col = ((s + KLAG) % gps) * G

                    @pl.when(k > 0)
                    def _():
                        retire(sr, tr, row, col)
            return carry

        lax.fori_loop(0, npass, pass_body, 0)

        for e in range(KLAG):
            tr = ng - KLAG + e
            s2 = tr % NSLOT
            retire(s2, tr, row0 + tr // gps, (tr % gps) * G)
        for s in range(NSLOT):
            wait_scatter(s)

    return emb(table, idx, pe)


def kernel(tokens, table):
    b, s = tokens.shape
    v, d = table.shape
    assert (b * s) % (NW * G * NSLOT) == 0 and s % G == 0 and d % L == 0
    pe = jnp.asarray(_pe_table(s, d))
    return _build(tokens, table, pe, seq=s, d=d)
